# baseline (speedup 1.0000x reference)
import jax
import jax.numpy as jnp
from jax.experimental import pallas as pl
from jax.experimental.pallas import tpu as pltpu

N = 64
E = 128
H = 128
NUM_MSG = 6
NUM_ATOM_TYPES = 171
NUM_BOND_TYPES = 200
OUT_DIM = 2
NB = E // 2
BM = 16

_BF = jnp.bfloat16
_DN = (((1,), (0,)), ((), ()))
_DNT = (((1,), (1,)), ((), ()))


def _dot(a, b):
    return jax.lax.dot_general(a.astype(_BF), b, _DN,
                               preferred_element_type=jnp.float32)


def _dotT(a, b):
    return jax.lax.dot_general(a.astype(_BF), b, _DNT,
                               preferred_element_type=jnp.float32)


def _make_body(n_edge, n_node):
    def body(atom3_r, bond3_r, conn_r, bidx_r,
             aemb_r, bemb_r, memb_r, bdew_r, *rest):
        ws = list(rest[:-1])
        out = rest[-1]
        groups = []
        for cnt in (n_edge,) * 4 + (n_node,) * 8:
            groups.append([r[...] for r in ws[:cnt]])
            ws = ws[cnt:]
        EW1, EB1, EW2, EB2, MW1, MB1, MW2, MB2, UW1, UB1, UW2, UB2 = groups

        f32 = jnp.float32
        relu = lambda x: jnp.maximum(x, 0.0)

        aemb = aemb_r[...]
        bemb = bemb_r[...]
        memb = memb_r[...]
        bdew = bdew_r[...]

        A3 = atom3_r[...]
        B3 = bond3_r[...]
        C3 = conn_r[...]
        BI = bidx_r[...]

        iota_an = jax.lax.broadcasted_iota(jnp.int32, (N, NUM_ATOM_TYPES), 1)
        iota_bn = jax.lax.broadcasted_iota(jnp.int32, (E, NUM_BOND_TYPES), 1)
        iota_en = jax.lax.broadcasted_iota(jnp.int32, (E, N), 1)
        iota_ne = jax.lax.broadcasted_iota(jnp.int32, (N, E), 0)
        iota_be = jax.lax.broadcasted_iota(jnp.int32, (NB, E), 0)

        atom_parts, bond_parts = [], []
        amask_parts, bmask_parts = [], []
        src_g, dst_g, src_s, agg_oh, mean_lk = [], [], [], [], []
        for m in range(BM):
            a = A3[m]
            b = B3[m]
            s = C3[m][:, 0:1]
            d = C3[m][:, 1:2]
            bi = BI[m:m + 1, :]
            a_oh = (iota_an == a).astype(_BF)
            b_oh = (iota_bn == b).astype(_BF)
            atom_parts.append(_dot(a_oh, aemb))
            bond_parts.append(_dot(b_oh, bemb))
            mean_lk.append(_dot(b_oh, memb))
            amask_parts.append((a != 0).astype(f32))
            bmask_parts.append((b != 0).astype(f32))
            src_g.append((iota_en == s).astype(_BF))
            dst_g.append((iota_en == d).astype(_BF))
            src_s.append((iota_ne == s.T).astype(_BF))
            agg_oh.append((iota_be == bi).astype(_BF))

        atom_state = jnp.concatenate(atom_parts, axis=0)
        bond_state = jnp.concatenate(bond_parts, axis=0)
        amask = jnp.concatenate(amask_parts, axis=0)
        bmask = jnp.concatenate(bmask_parts, axis=0)

        for i in range(NUM_MSG):
            W1 = EW1[i]
            src_atom = jnp.concatenate(
                [_dot(src_g[m], atom_state[m * N:(m + 1) * N].astype(_BF))
                 for m in range(BM)], axis=0)
            dst_atom = jnp.concatenate(
                [_dot(dst_g[m], atom_state[m * N:(m + 1) * N].astype(_BF))
                 for m in range(BM)], axis=0)
            h = relu(_dotT(bond_state, W1[:, 0:H])
                     + _dotT(src_atom, W1[:, H:2 * H])
                     + _dotT(dst_atom, W1[:, 2 * H:3 * H])
                     + EB1[i])
            nb = _dotT(h, EW2[i]) + EB2[i]
            bond_state = bond_state + nb * bmask
            if i < NUM_MSG - 1:
                M1 = MW1[i]
                h2 = relu(_dotT(dst_atom, M1[:, 0:H])
                          + _dotT(bond_state, M1[:, H:2 * H]) + MB1[i])
                msg = (_dotT(h2, MW2[i]) + MB2[i]) * bmask
                agg = jnp.concatenate(
                    [_dot(src_s[m], msg[m * E:(m + 1) * E].astype(_BF))
                     for m in range(BM)], axis=0)
                na = relu(_dotT(agg, UW1[i]) + UB1[i])
                na = _dotT(na, UW2[i]) + UB2[i]
                atom_state = atom_state + na * amask

        masked = bond_state * bmask
        for m in range(BM):
            msl = masked[m * E:(m + 1) * E].astype(_BF)
            feat = _dot(agg_oh[m], msl)
            cnt = jnp.maximum(_dot(agg_oh[m], bmask_parts[m].astype(_BF)), 1.0)
            magg = _dot(agg_oh[m], (mean_lk[m] * bmask_parts[m]))
            out[m] = _dotT(feat / cnt, bdew) + magg / cnt

    return body


@jax.jit
def kernel(atom, bond, connectivity, bond_indices, params):
    B = atom.shape[0]
    atom3 = atom.astype(jnp.int32).reshape(B, N, 1)
    bond3 = bond.astype(jnp.int32).reshape(B, E, 1)
    conn = connectivity.astype(jnp.int32)
    bidx = bond_indices.astype(jnp.int32)

    bf = lambda x: x.astype(_BF)
    ew1 = [bf(p['w1']) for p in params['edge']]
    eb1 = [p['b1'].reshape(1, -1) for p in params['edge']]
    ew2 = [bf(p['w2']) for p in params['edge']]
    eb2 = [p['b2'].reshape(1, -1) for p in params['edge']]
    mw1 = [bf(p['mw1']) for p in params['node']]
    mb1 = [p['mb1'].reshape(1, -1) for p in params['node']]
    mw2 = [bf(p['mw2']) for p in params['node']]
    mb2 = [p['mb2'].reshape(1, -1) for p in params['node']]
    uw1 = [bf(p['uw1']) for p in params['node']]
    ub1 = [p['ub1'].reshape(1, -1) for p in params['node']]
    uw2 = [bf(p['uw2']) for p in params['node']]
    ub2 = [p['ub2'].reshape(1, -1) for p in params['node']]
    bdew = bf(params['bde_no_mean_w'])
    aemb = bf(params['atom_emb'])
    bemb = bf(params['bond_emb'])
    memb = bf(params['bde_mean_emb'])

    weight_list = (ew1 + eb1 + ew2 + eb2 + mw1 + mb1 + mw2 + mb2
                   + uw1 + ub1 + uw2 + ub2)

    grid = (B // BM,)
    blk = lambda *shape: pl.BlockSpec(shape, lambda i: (i,) + (0,) * (len(shape) - 1))
    full = lambda a: pl.BlockSpec(a.shape, lambda i: (0,) * a.ndim)

    out = pl.pallas_call(
        _make_body(len(ew1), len(mw1)),
        grid=grid,
        in_specs=[blk(BM, N, 1), blk(BM, E, 1), blk(BM, E, 2), blk(BM, E),
                  full(aemb), full(bemb), full(memb), full(bdew)]
                 + [full(w) for w in weight_list],
        out_specs=blk(BM, NB, OUT_DIM),
        out_shape=jax.ShapeDtypeStruct((B, NB, OUT_DIM), jnp.float32),
        compiler_params=pltpu.CompilerParams(
            dimension_semantics=("arbitrary",),
        ),
    )(atom3, bond3, conn, bidx, aemb, bemb, memb, bdew, *weight_list)
    return out


# speedup vs baseline: 1.2948x; 1.2948x over previous
import jax
import jax.numpy as jnp
from jax.experimental import pallas as pl
from jax.experimental.pallas import tpu as pltpu

N = 64
E = 128
H = 128
NUM_MSG = 6
NUM_ATOM_TYPES = 171
NUM_BOND_TYPES = 200
OUT_DIM = 2
NB = E // 2
BM = 16
GG = 4
GS = 2

_BF = jnp.bfloat16
_F32 = jnp.float32
_DN = (((1,), (0,)), ((), ()))
_DNT = (((1,), (1,)), ((), ()))


def _dot(a, b):
    return jax.lax.dot_general(a, b, _DN, preferred_element_type=_F32)


def _dotT(a, b):
    return jax.lax.dot_general(a, b, _DNT, preferred_element_type=_F32)


def _make_body(n_edge, n_node):
    def body(atom3_r, bond3_r, conn_r, bidx_r,
             aemb_r, bemb_r, memb_r, bdew_r, *rest):
        ws = list(rest[:-1])
        out = rest[-1]
        groups = []
        for cnt in (n_edge,) * 4 + (n_node,) * 8:
            groups.append([r[...] for r in ws[:cnt]])
            ws = ws[cnt:]
        EW1, EB1, EW2, EB2, MW1, MB1, MW2, MB2, UW1, UB1, UW2, UB2 = groups

        relu = lambda x: jnp.maximum(x, 0)

        aemb = aemb_r[...]
        bemb = bemb_r[...]
        memb = memb_r[...]
        bdew = bdew_r[...]

        A3 = atom3_r[...]
        B3 = bond3_r[...]
        C3 = conn_r[...]
        BI = bidx_r[...]

        iota_an = jax.lax.broadcasted_iota(jnp.int32, (N, NUM_ATOM_TYPES), 1)
        iota_bn = jax.lax.broadcasted_iota(jnp.int32, (E, NUM_BOND_TYPES), 1)
        iota_g = jax.lax.broadcasted_iota(jnp.int32, (GG * E, GG * N), 1)
        iota_s = jax.lax.broadcasted_iota(jnp.int32, (GS * N, GS * E), 0)
        iota_a = jax.lax.broadcasted_iota(jnp.int32, (GS * NB, GS * E), 0)

        atom_parts, bond_parts = [], []
        amask_parts, bmask_parts = [], []
        svec, dvec, bivec, mean_lk = [], [], [], []
        for m in range(BM):
            a = A3[m]
            b = B3[m]
            svec.append(C3[m][:, 0:1])
            dvec.append(C3[m][:, 1:2])
            bivec.append(BI[m:m + 1, :])
            a_oh = (iota_an == a).astype(_BF)
            b_oh = (iota_bn == b).astype(_BF)
            atom_parts.append(_dot(a_oh, aemb))
            bond_parts.append(_dot(b_oh, bemb))
            mean_lk.append(_dot(b_oh, memb))
            amask_parts.append((a != 0).astype(_F32))
            bmask_parts.append((b != 0).astype(_F32))

        src_g, dst_g = [], []
        for g in range(BM // GG):
            gs = jnp.concatenate(
                [svec[g * GG + j] + j * N for j in range(GG)], axis=0)
            gd = jnp.concatenate(
                [dvec[g * GG + j] + j * N for j in range(GG)], axis=0)
            src_g.append((iota_g == gs).astype(_BF))
            dst_g.append((iota_g == gd).astype(_BF))
        src_s, agg_oh = [], []
        for g in range(BM // GS):
            gs = jnp.concatenate(
                [svec[g * GS + j].T + j * N for j in range(GS)], axis=1)
            gb = jnp.concatenate(
                [bivec[g * GS + j] + j * NB for j in range(GS)], axis=1)
            src_s.append((iota_s == gs).astype(_BF))
            agg_oh.append((iota_a == gb).astype(_BF))

        atom_state = jnp.concatenate(atom_parts, axis=0)
        bond_state = jnp.concatenate(bond_parts, axis=0)
        amask = jnp.concatenate(amask_parts, axis=0)
        bmask = jnp.concatenate(bmask_parts, axis=0)
        bmask_b = bmask.astype(_BF)

        GN, GE = GG * N, GG * E
        SN, SE = GS * N, GS * E
        for i in range(NUM_MSG):
            W1 = EW1[i]
            ab = atom_state.astype(_BF)
            bb = bond_state.astype(_BF)
            src_atom = jnp.concatenate(
                [_dot(src_g[g], ab[g * GN:(g + 1) * GN]).astype(_BF)
                 for g in range(BM // GG)], axis=0)
            dst_atom = jnp.concatenate(
                [_dot(dst_g[g], ab[g * GN:(g + 1) * GN]).astype(_BF)
                 for g in range(BM // GG)], axis=0)
            x1 = jnp.concatenate([bb, src_atom, dst_atom], axis=1)
            h = relu(_dotT(x1, W1) + EB1[i]).astype(_BF)
            nb = _dotT(h, EW2[i]) + EB2[i]
            bond_state = bond_state + nb * bmask
            if i < NUM_MSG - 1:
                M1 = MW1[i]
                bb2 = bond_state.astype(_BF)
                x2 = jnp.concatenate([dst_atom, bb2], axis=1)
                h2 = relu(_dotT(x2, M1) + MB1[i]).astype(_BF)
                msg = ((_dotT(h2, MW2[i]) + MB2[i]).astype(_BF)
                       * bmask_b)
                agg = jnp.concatenate(
                    [_dot(src_s[g], msg[g * SE:(g + 1) * SE]).astype(_BF)
                     for g in range(BM // GS)], axis=0)
                na = relu(_dotT(agg, UW1[i]) + UB1[i]).astype(_BF)
                na2 = _dotT(na, UW2[i]) + UB2[i]
                atom_state = atom_state + na2 * amask

        masked = (bond_state * bmask).astype(_BF)
        for g in range(BM // GS):
            msl = masked[g * SE:(g + 1) * SE]
            bm2 = jnp.concatenate(
                [bmask_parts[g * GS + j] for j in range(GS)], axis=0)
            ml2 = jnp.concatenate(
                [mean_lk[g * GS + j] * bmask_parts[g * GS + j]
                 for j in range(GS)], axis=0)
            feat = _dot(agg_oh[g], msl)
            cnt = jnp.maximum(_dot(agg_oh[g], bm2.astype(_BF)), 1.0)
            magg = _dot(agg_oh[g], ml2.astype(_BF))
            res = _dotT((feat / cnt).astype(_BF), bdew) + magg / cnt
            for j in range(GS):
                out[g * GS + j] = res[j * NB:(j + 1) * NB]

    return body


@jax.jit
def kernel(atom, bond, connectivity, bond_indices, params):
    B = atom.shape[0]
    atom3 = atom.astype(jnp.int32).reshape(B, N, 1)
    bond3 = bond.astype(jnp.int32).reshape(B, E, 1)
    conn = connectivity.astype(jnp.int32)
    bidx = bond_indices.astype(jnp.int32)

    bf = lambda x: x.astype(_BF)
    ew1 = [bf(p['w1']) for p in params['edge']]
    eb1 = [bf(p['b1']).reshape(1, -1) for p in params['edge']]
    ew2 = [bf(p['w2']) for p in params['edge']]
    eb2 = [p['b2'].reshape(1, -1) for p in params['edge']]
    mw1 = [bf(p['mw1']) for p in params['node']]
    mb1 = [bf(p['mb1']).reshape(1, -1) for p in params['node']]
    mw2 = [bf(p['mw2']) for p in params['node']]
    mb2 = [bf(p['mb2']).reshape(1, -1) for p in params['node']]
    uw1 = [bf(p['uw1']) for p in params['node']]
    ub1 = [bf(p['ub1']).reshape(1, -1) for p in params['node']]
    uw2 = [bf(p['uw2']) for p in params['node']]
    ub2 = [p['ub2'].reshape(1, -1) for p in params['node']]
    bdew = bf(params['bde_no_mean_w'])
    aemb = bf(params['atom_emb'])
    bemb = bf(params['bond_emb'])
    memb = bf(params['bde_mean_emb'])

    weight_list = (ew1 + eb1 + ew2 + eb2 + mw1 + mb1 + mw2 + mb2
                   + uw1 + ub1 + uw2 + ub2)

    grid = (B // BM,)
    blk = lambda *shape: pl.BlockSpec(shape, lambda i: (i,) + (0,) * (len(shape) - 1))
    full = lambda a: pl.BlockSpec(a.shape, lambda i: (0,) * a.ndim)

    out = pl.pallas_call(
        _make_body(len(ew1), len(mw1)),
        grid=grid,
        in_specs=[blk(BM, N, 1), blk(BM, E, 1), blk(BM, E, 2), blk(BM, E),
                  full(aemb), full(bemb), full(memb), full(bdew)]
                 + [full(w) for w in weight_list],
        out_specs=blk(BM, NB, OUT_DIM),
        out_shape=jax.ShapeDtypeStruct((B, NB, OUT_DIM), jnp.float32),
        compiler_params=pltpu.CompilerParams(
            dimension_semantics=("arbitrary",),
        ),
    )(atom3, bond3, conn, bidx, aemb, bemb, memb, bdew, *weight_list)
    return out


# speedup vs baseline: 1.4105x; 1.0893x over previous
import jax
import jax.numpy as jnp
from jax.experimental import pallas as pl
from jax.experimental.pallas import tpu as pltpu

N = 64
E = 128
H = 128
NUM_MSG = 6
NUM_ATOM_TYPES = 171
NUM_BOND_TYPES = 200
OUT_DIM = 2
NB = E // 2
BM = 16
GG = 4
GS = 2

_BF = jnp.bfloat16
_F32 = jnp.float32
_DN = (((1,), (0,)), ((), ()))
_DNT = (((1,), (1,)), ((), ()))


def _dot(a, b):
    return jax.lax.dot_general(a, b, _DN, preferred_element_type=_F32)


def _dotT(a, b):
    return jax.lax.dot_general(a, b, _DNT, preferred_element_type=_F32)


def _make_body(n_edge, n_node):
    def body(atom3_r, bond3_r, conn_r, bidx_r,
             aemb_r, bemb_r, bdew_r, *rest):
        ws = list(rest[:-1])
        out = rest[-1]
        groups = []
        for cnt in (n_edge,) * 4 + (n_node,) * 8:
            groups.append([r[...] for r in ws[:cnt]])
            ws = ws[cnt:]
        EW1, EB1, EW2, EB2, MW1, MB1, MW2, MB2, UW1, UB1, UW2, UB2 = groups

        relu = lambda x: jnp.maximum(x, 0)

        aemb = aemb_r[...]
        bemb = bemb_r[...]
        bdew = bdew_r[...]

        A3 = atom3_r[...]
        B3 = bond3_r[...]
        C3 = conn_r[...]
        BI = bidx_r[...]

        iota_an = jax.lax.broadcasted_iota(jnp.int32, (BM * N, NUM_ATOM_TYPES), 1)
        iota_bn = jax.lax.broadcasted_iota(jnp.int32, (BM * E, NUM_BOND_TYPES), 1)
        iota_g = jax.lax.broadcasted_iota(jnp.int32, (GG * E, GG * N), 1)
        iota_s = jax.lax.broadcasted_iota(jnp.int32, (GS * N, GS * E), 0)
        iota_a = jax.lax.broadcasted_iota(jnp.int32, (GS * NB, GS * E), 0)

        svec, dvec, bivec = [], [], []
        for m in range(BM):
            svec.append(C3[m][:, 0:1])
            dvec.append(C3[m][:, 1:2])
            bivec.append(BI[m:m + 1, :])

        a_all = jnp.concatenate([A3[m] for m in range(BM)], axis=0)
        b_all = jnp.concatenate([B3[m] for m in range(BM)], axis=0)
        a_oh = (iota_an == a_all).astype(_BF)
        b_oh = (iota_bn == b_all).astype(_BF)
        atom_state = _dot(a_oh, aemb)
        bond_init = _dot(b_oh, bemb)
        bond_state = bond_init[:, 0:H]
        mean_lk = bond_init[:, H:H + OUT_DIM]
        amask = (a_all != 0).astype(_F32)
        bmask = (b_all != 0).astype(_F32)

        src_g, dst_g = [], []
        for g in range(BM // GG):
            gs = jnp.concatenate(
                [svec[g * GG + j] + j * N for j in range(GG)], axis=0)
            gd = jnp.concatenate(
                [dvec[g * GG + j] + j * N for j in range(GG)], axis=0)
            src_g.append((iota_g == gs).astype(_BF))
            dst_g.append((iota_g == gd).astype(_BF))
        src_s, agg_oh = [], []
        for g in range(BM // GS):
            gs = jnp.concatenate(
                [svec[g * GS + j].T + j * N for j in range(GS)], axis=1)
            gb = jnp.concatenate(
                [bivec[g * GS + j] + j * NB for j in range(GS)], axis=1)
            src_s.append((iota_s == gs).astype(_BF))
            agg_oh.append((iota_a == gb).astype(_BF))

        bmask_b = bmask.astype(_BF)

        GN, GE = GG * N, GG * E
        SN, SE = GS * N, GS * E
        for i in range(NUM_MSG):
            W1 = EW1[i]
            ab = atom_state.astype(_BF)
            bb = bond_state.astype(_BF)
            src_atom = jnp.concatenate(
                [_dot(src_g[g], ab[g * GN:(g + 1) * GN]).astype(_BF)
                 for g in range(BM // GG)], axis=0)
            dst_atom = jnp.concatenate(
                [_dot(dst_g[g], ab[g * GN:(g + 1) * GN]).astype(_BF)
                 for g in range(BM // GG)], axis=0)
            x1 = jnp.concatenate([bb, src_atom, dst_atom], axis=1)
            h = relu(_dotT(x1, W1).astype(_BF) + EB1[i])
            nb = _dotT(h, EW2[i]) + EB2[i]
            bond_state = bond_state + nb * bmask
            if i < NUM_MSG - 1:
                M1 = MW1[i]
                bb2 = bond_state.astype(_BF)
                x2 = jnp.concatenate([dst_atom, bb2], axis=1)
                h2 = relu(_dotT(x2, M1).astype(_BF) + MB1[i])
                msg = ((_dotT(h2, MW2[i]).astype(_BF) + MB2[i])
                       * bmask_b)
                agg = jnp.concatenate(
                    [_dot(src_s[g], msg[g * SE:(g + 1) * SE]).astype(_BF)
                     for g in range(BM // GS)], axis=0)
                na = relu(_dotT(agg, UW1[i]).astype(_BF) + UB1[i])
                na2 = _dotT(na, UW2[i]) + UB2[i]
                atom_state = atom_state + na2 * amask

        payload = jnp.concatenate(
            [bond_state * bmask, bmask, mean_lk * bmask],
            axis=1).astype(_BF)
        feats, cnts, maggs = [], [], []
        for g in range(BM // GS):
            r = _dot(agg_oh[g], payload[g * SE:(g + 1) * SE])
            feats.append(r[:, 0:H])
            cnts.append(r[:, H:H + 1])
            maggs.append(r[:, H + 1:H + 1 + OUT_DIM])
        feat = jnp.concatenate(feats, axis=0)
        cnt = jnp.maximum(jnp.concatenate(cnts, axis=0), 1.0)
        magg = jnp.concatenate(maggs, axis=0)
        res = _dotT((feat / cnt).astype(_BF), bdew) + magg / cnt
        for m in range(BM):
            out[m] = res[m * NB:(m + 1) * NB]

    return body


@jax.jit
def kernel(atom, bond, connectivity, bond_indices, params):
    B = atom.shape[0]
    atom3 = atom.astype(jnp.int32).reshape(B, N, 1)
    bond3 = bond.astype(jnp.int32).reshape(B, E, 1)
    conn = connectivity.astype(jnp.int32)
    bidx = bond_indices.astype(jnp.int32)

    bf = lambda x: x.astype(_BF)
    ew1 = [bf(p['w1']) for p in params['edge']]
    eb1 = [bf(p['b1']).reshape(1, -1) for p in params['edge']]
    ew2 = [bf(p['w2']) for p in params['edge']]
    eb2 = [p['b2'].reshape(1, -1) for p in params['edge']]
    mw1 = [bf(p['mw1']) for p in params['node']]
    mb1 = [bf(p['mb1']).reshape(1, -1) for p in params['node']]
    mw2 = [bf(p['mw2']) for p in params['node']]
    mb2 = [bf(p['mb2']).reshape(1, -1) for p in params['node']]
    uw1 = [bf(p['uw1']) for p in params['node']]
    ub1 = [bf(p['ub1']).reshape(1, -1) for p in params['node']]
    uw2 = [bf(p['uw2']) for p in params['node']]
    ub2 = [p['ub2'].reshape(1, -1) for p in params['node']]
    bdew = bf(params['bde_no_mean_w'])
    aemb = bf(params['atom_emb'])
    bemb = bf(jnp.concatenate(
        [params['bond_emb'], params['bde_mean_emb']], axis=1))

    weight_list = (ew1 + eb1 + ew2 + eb2 + mw1 + mb1 + mw2 + mb2
                   + uw1 + ub1 + uw2 + ub2)

    grid = (B // BM,)
    blk = lambda *shape: pl.BlockSpec(shape, lambda i: (i,) + (0,) * (len(shape) - 1))
    full = lambda a: pl.BlockSpec(a.shape, lambda i: (0,) * a.ndim)

    out = pl.pallas_call(
        _make_body(len(ew1), len(mw1)),
        grid=grid,
        in_specs=[blk(BM, N, 1), blk(BM, E, 1), blk(BM, E, 2), blk(BM, E),
                  full(aemb), full(bemb), full(bdew)]
                 + [full(w) for w in weight_list],
        out_specs=blk(BM, NB, OUT_DIM),
        out_shape=jax.ShapeDtypeStruct((B, NB, OUT_DIM), jnp.float32),
        compiler_params=pltpu.CompilerParams(
            dimension_semantics=("arbitrary",),
        ),
    )(atom3, bond3, conn, bidx, aemb, bemb, bdew, *weight_list)
    return out
